# baseline (speedup 1.0000x reference)
import dataclasses
import functools

import jax
import jax.numpy as jnp
from jax import lax
from jax.experimental import pallas as pl
from jax.experimental.pallas import tpu as pltpu
from jax.experimental.pallas import tpu_sc as plsc

_N = 10000
_NP = 10240
_H = 16
_C = 2
_NCORE = 2
_NSUB = 16
_NW = _NCORE * _NSUB
_EP = 327680
_EPW = _EP // _NW
_CB = 1024
_NCH = _EPW // _CB
_NPT = _NP // _NSUB

_mesh = plsc.VectorSubcoreMesh(core_axis_name="c", subcore_axis_name="s")

_sc_cp = pltpu.CompilerParams()
if "needs_layout_passes" in pltpu.CompilerParams.__dataclass_fields__:
  _sc_cp = dataclasses.replace(_sc_cp, needs_layout_passes=False)
_sc_cp = dataclasses.replace(_sc_cp, use_tc_tiling_on_sc=False)


def _sc_deg(dst3d, ew3d, zeros_nh):
  @functools.partial(
      pl.kernel,
      out_type=jax.ShapeDtypeStruct((_NCORE, _NP, _H), jnp.float32),
      mesh=_mesh,
      compiler_params=_sc_cp,
      scratch_types=[
          pltpu.VMEM((_NCH, _CB), jnp.int32),
          pltpu.VMEM((_NCH, _CB), jnp.float32),
          pltpu.VMEM((_CB, _H), jnp.float32),
          pltpu.VMEM((_CB, _H), jnp.float32),
          pltpu.VMEM_SHARED((_NP, _H), jnp.float32),
          pltpu.SemaphoreType.DMA,
          pltpu.SemaphoreType.DMA,
      ],
  )
  def k(dst_hbm, ew_hbm, z_hbm, out_hbm, dstv, ewv, rows0, rows1, acc_sh,
        ss0, ss1):
    c = lax.axis_index("c")
    s = lax.axis_index("s")
    w = c * _NSUB + s
    pltpu.sync_copy(z_hbm.at[pl.ds(s * _NPT, _NPT)],
                    acc_sh.at[pl.ds(s * _NPT, _NPT)])
    pltpu.sync_copy(dst_hbm.at[w], dstv)
    pltpu.sync_copy(ew_hbm.at[w], ewv)
    zrow = jnp.zeros((_H,), jnp.float32)
    bufs = (rows0, rows1)
    sems = (ss0, ss1)
    pend = [None, None]

    for rowsv in bufs:
      @pl.loop(0, _CB)
      def _(i):
        rowsv[i, :] = zrow

    plsc.subcore_barrier()
    iota = lax.iota(jnp.int32, 16)
    zer16 = jnp.zeros((16,), jnp.int32)

    for ch in range(_NCH):
      b = ch % 2
      rowsv = bufs[b]
      if pend[b] is not None:
        pend[b].wait()
      cful = jnp.full((16,), ch, jnp.int32)

      @pl.loop(0, _CB // 16)
      def _(g):
        ew16 = plsc.load_gather(ewv, [cful, iota + g * 16])
        plsc.store_scatter(rowsv, [iota + g * 16, zer16], ew16)

      pend[b] = pltpu.async_copy(rowsv, acc_sh.at[dstv.at[ch]],
                                 sems[b], add=True)

    for p in pend:
      p.wait()

    plsc.subcore_barrier()
    pltpu.sync_copy(acc_sh.at[pl.ds(s * _NPT, _NPT)],
                    out_hbm.at[c, pl.ds(s * _NPT, _NPT)])

  return k(dst3d, ew3d, zeros_nh)


def _sc_agg(src3d, dst3d, ew3d, table_nh, zeros_nh):
  @functools.partial(
      pl.kernel,
      out_type=jax.ShapeDtypeStruct((_NCORE, _NP, _H), jnp.float32),
      mesh=_mesh,
      compiler_params=_sc_cp,
      scratch_types=[
          pltpu.VMEM((_NCH, _CB), jnp.int32),
          pltpu.VMEM((_NCH, _CB), jnp.int32),
          pltpu.VMEM((_NCH, _CB), jnp.float32),
          pltpu.VMEM((_CB, _H), jnp.float32),
          pltpu.VMEM((_CB, _H), jnp.float32),
          pltpu.VMEM((_CB, _H), jnp.float32),
          pltpu.VMEM((_CB, _H), jnp.float32),
          pltpu.VMEM_SHARED((_NP, _H), jnp.float32),
          pltpu.SemaphoreType.DMA,
          pltpu.SemaphoreType.DMA,
          pltpu.SemaphoreType.DMA,
          pltpu.SemaphoreType.DMA,
          pltpu.SemaphoreType.DMA,
          pltpu.SemaphoreType.DMA,
          pltpu.SemaphoreType.DMA,
          pltpu.SemaphoreType.DMA,
      ],
  )
  def k(src_hbm, dst_hbm, ew_hbm, tab_hbm, z_hbm,
        out_hbm, srcv, dstv, ewv, rows0, rows1, rows2, rows3, acc_sh,
        gs0, gs1, gs2, gs3, ss0, ss1, ss2, ss3):
    c = lax.axis_index("c")
    s = lax.axis_index("s")
    w = c * _NSUB + s
    pltpu.sync_copy(z_hbm.at[pl.ds(s * _NPT, _NPT)],
                    acc_sh.at[pl.ds(s * _NPT, _NPT)])
    pltpu.sync_copy(src_hbm.at[w], srcv)
    pltpu.sync_copy(dst_hbm.at[w], dstv)
    pltpu.sync_copy(ew_hbm.at[w], ewv)
    plsc.subcore_barrier()

    bufs = (rows0, rows1, rows2, rows3)
    gsems = (gs0, gs1, gs2, gs3)
    ssems = (ss0, ss1, ss2, ss3)
    pg = [None] * 4
    ps = [None] * 4
    for g in range(3):
      pg[g] = pltpu.async_copy(tab_hbm.at[srcv.at[g]], bufs[g], gsems[g])
    for ch in range(_NCH):
      b = ch % 4
      rows = bufs[b]
      pg[b].wait()
      cful = jnp.full((16,), ch, jnp.int32)

      @pl.loop(0, _CB // 4)
      def _(q):
        i0 = q * 4
        for u in range(4):
          sp = plsc.load_gather(
              ewv, [cful, jnp.full((16,), i0 + u, jnp.int32)])
          rows[i0 + u, :] = rows[i0 + u, :] * sp

      ps[b] = pltpu.async_copy(rows, acc_sh.at[dstv.at[ch]],
                               ssems[b], add=True)
      g = ch + 3
      if g < _NCH:
        gb = g % 4
        if ps[gb] is not None:
          ps[gb].wait()
        pg[gb] = pltpu.async_copy(tab_hbm.at[srcv.at[g]], bufs[gb],
                                  gsems[gb])
    for ch in range(_NCH - 4, _NCH):
      ps[ch % 4].wait()

    plsc.subcore_barrier()
    pltpu.sync_copy(acc_sh.at[pl.ds(s * _NPT, _NPT)],
                    out_hbm.at[c, pl.ds(s * _NPT, _NPT)])

  return k(src3d, dst3d, ew3d, table_nh, zeros_nh)


_NPK = _NP // 8


def _tc_h1p(x4, W1e):
  def body(x_ref, w_ref, o_ref):
    acc = jnp.zeros((_NPK, 128), jnp.float32)
    for k in range(8):
      acc = acc + jnp.dot(x_ref[:, k, :], w_ref[k],
                          preferred_element_type=jnp.float32)
    o_ref[...] = acc

  return pl.pallas_call(
      body,
      in_specs=[pl.BlockSpec((_NPK, 8, 128), lambda: (0, 0, 0)),
                pl.BlockSpec((8, 128, 128), lambda: (0, 0, 0))],
      out_specs=pl.BlockSpec((_NPK, 128), lambda: (0, 0)),
      out_shape=jax.ShapeDtypeStruct((_NPK, 128), jnp.float32))(x4, W1e)


def _tc_post_deg(partsp, h1p, S, B):
  def body(p_ref, h_ref, s_ref, b_ref, xs_ref, d_ref):
    deg8 = jnp.dot(p_ref[0] + p_ref[1], s_ref[...],
                   preferred_element_type=jnp.float32,
                   precision=lax.Precision.HIGHEST) + 1.0
    d = jnp.dot(lax.rsqrt(deg8), b_ref[...],
                preferred_element_type=jnp.float32,
                precision=lax.Precision.HIGHEST)
    d_ref[...] = d
    xs_ref[...] = h_ref[...] * d

  return pl.pallas_call(
      body,
      in_specs=[pl.BlockSpec((_NCORE, _NPK, 128), lambda: (0, 0, 0)),
                pl.BlockSpec((_NPK, 128), lambda: (0, 0)),
                pl.BlockSpec((128, 8), lambda: (0, 0)),
                pl.BlockSpec((8, 128), lambda: (0, 0))],
      out_specs=[pl.BlockSpec((_NPK, 128), lambda: (0, 0))] * 2,
      out_shape=[jax.ShapeDtypeStruct((_NPK, 128), jnp.float32)] * 2,
  )(partsp, h1p, S, B)


def _tc_mid(partsp, dinv, h1p, b1t):
  def body(p_ref, d_ref, h_ref, b_ref, rs_ref, r_ref):
    d = d_ref[...]
    a1 = d * (p_ref[0] + p_ref[1]) + d * d * h_ref[...] + b_ref[...]
    r = jnp.maximum(a1, 0.0)
    r_ref[...] = r
    rs_ref[...] = d * r

  return pl.pallas_call(
      body,
      in_specs=[pl.BlockSpec((_NCORE, _NPK, 128), lambda: (0, 0, 0)),
                pl.BlockSpec((_NPK, 128), lambda: (0, 0)),
                pl.BlockSpec((_NPK, 128), lambda: (0, 0)),
                pl.BlockSpec((1, 128), lambda: (0, 0))],
      out_specs=[pl.BlockSpec((_NPK, 128), lambda: (0, 0))] * 2,
      out_shape=[jax.ShapeDtypeStruct((_NPK, 128), jnp.float32)] * 2,
  )(partsp, dinv, h1p, b1t)


def _tc_final(partsp, dinv, rp, W2e, P01, b2):
  def body(p_ref, d_ref, r_ref, w_ref, q_ref, b_ref, o_ref):
    d = d_ref[...]
    a2 = d * (p_ref[0] + p_ref[1]) + d * d * r_ref[...]
    z0 = jnp.dot(a2, w_ref[0], preferred_element_type=jnp.float32) + b_ref[0, 0]
    z1 = jnp.dot(a2, w_ref[1], preferred_element_type=jnp.float32) + b_ref[0, 1]
    m = jnp.maximum(z0, z1)
    lse = m + jnp.log(jnp.exp(z0 - m) + jnp.exp(z1 - m))
    o_ref[...] = (jnp.dot(z0 - lse, q_ref[0],
                          preferred_element_type=jnp.float32,
                          precision=lax.Precision.HIGHEST)
                  + jnp.dot(z1 - lse, q_ref[1],
                            preferred_element_type=jnp.float32,
                            precision=lax.Precision.HIGHEST))

  return pl.pallas_call(
      body,
      in_specs=[pl.BlockSpec((_NCORE, _NPK, 128), lambda: (0, 0, 0)),
                pl.BlockSpec((_NPK, 128), lambda: (0, 0)),
                pl.BlockSpec((_NPK, 128), lambda: (0, 0)),
                pl.BlockSpec((2, 128, 8), lambda: (0, 0, 0)),
                pl.BlockSpec((2, 8, 16), lambda: (0, 0, 0)),
                pl.BlockSpec((1, 2), lambda: (0, 0))],
      out_specs=pl.BlockSpec((_NPK, 16), lambda: (0, 0)),
      out_shape=jax.ShapeDtypeStruct((_NPK, 16), jnp.float32),
  )(partsp, dinv, rp, W2e, P01, b2)


def kernel(x, edge_index, edge_weight, W1, b1, W2, b2):
  src = edge_index[0].astype(jnp.int32)
  dst = edge_index[1].astype(jnp.int32)
  ew = edge_weight.astype(jnp.float32)
  pad = _EP - ew.shape[0]
  src3d = jnp.pad(src, (0, pad)).reshape(_NW, _NCH, _CB)
  dst3d = jnp.pad(dst, (0, pad)).reshape(_NW, _NCH, _CB)
  ew3d = jnp.pad(ew, (0, pad)).reshape(_NW, _NCH, _CB)
  zeros_nh = jnp.zeros((_NP, _H), jnp.float32)

  lanes = jnp.arange(128)
  S = (lanes[:, None] == jnp.arange(8)[None, :] * 16).astype(jnp.float32)
  B = (lanes[None, :] // 16 == jnp.arange(8)[:, None]).astype(jnp.float32)
  W1e = jnp.zeros((8, 128, 128), jnp.float32)
  for k in range(8):
    W1e = W1e.at[k, :, 16 * k:16 * k + 16].set(W1)
  W2e = jnp.zeros((2, 128, 8), jnp.float32)
  for k in range(8):
    W2e = W2e.at[0, 16 * k:16 * k + 16, k].set(W2[:, 0])
    W2e = W2e.at[1, 16 * k:16 * k + 16, k].set(W2[:, 1])
  P01 = jnp.zeros((2, 8, 16), jnp.float32)
  for k in range(8):
    P01 = P01.at[0, k, 2 * k].set(1.0)
    P01 = P01.at[1, k, 2 * k + 1].set(1.0)
  b1t = jnp.tile(b1, 8).reshape(1, 128)
  b2t = b2.reshape(1, 2)

  xp = jnp.pad(x, ((0, _NP - x.shape[0]), (0, 0)))
  x4 = xp.reshape(_NPK, 8, 128)
  h1p = _tc_h1p(x4, W1e)
  degp = _sc_deg(dst3d, ew3d, zeros_nh)
  degpp = degp.reshape(_NCORE, _NPK, 128)
  xsp, dinv = _tc_post_deg(degpp, h1p, S, B)
  a1p = _sc_agg(src3d, dst3d, ew3d, xsp.reshape(_NP, _H), zeros_nh)
  rsp, rp = _tc_mid(a1p.reshape(_NCORE, _NPK, 128), dinv, h1p, b1t)
  a2p = _sc_agg(src3d, dst3d, ew3d, rsp.reshape(_NP, _H), zeros_nh)
  outp = _tc_final(a2p.reshape(_NCORE, _NPK, 128), dinv, rp, W2e, P01, b2t)
  return outp.reshape(_NP, _C)[:_N]


# speedup vs baseline: 1.0211x; 1.0211x over previous
import dataclasses
import functools

import jax
import jax.numpy as jnp
from jax import lax
from jax.experimental import pallas as pl
from jax.experimental.pallas import tpu as pltpu
from jax.experimental.pallas import tpu_sc as plsc

_N = 10000
_NP = 10240
_H = 16
_C = 2
_NCORE = 2
_NSUB = 16
_NW = _NCORE * _NSUB
_EP = 327680
_EPW = _EP // _NW
_CB = 1024
_NCH = _EPW // _CB
_NPT = _NP // _NSUB

_mesh = plsc.VectorSubcoreMesh(core_axis_name="c", subcore_axis_name="s")

_sc_cp = pltpu.CompilerParams()
if "needs_layout_passes" in pltpu.CompilerParams.__dataclass_fields__:
  _sc_cp = dataclasses.replace(_sc_cp, needs_layout_passes=False)
_sc_cp = dataclasses.replace(_sc_cp, use_tc_tiling_on_sc=False)


def _sc_deg(dst3d, ew3d, zeros_nh):
  @functools.partial(
      pl.kernel,
      out_type=jax.ShapeDtypeStruct((_NCORE, _NP, _H), jnp.float32),
      mesh=_mesh,
      compiler_params=_sc_cp,
      scratch_types=[
          pltpu.VMEM((_NCH, _CB), jnp.int32),
          pltpu.VMEM((_NCH, _CB), jnp.float32),
          pltpu.VMEM((_CB, _H), jnp.float32),
          pltpu.VMEM((_CB, _H), jnp.float32),
          pltpu.VMEM_SHARED((_NP, _H), jnp.float32),
          pltpu.SemaphoreType.DMA,
          pltpu.SemaphoreType.DMA,
      ],
  )
  def k(dst_hbm, ew_hbm, z_hbm, out_hbm, dstv, ewv, rows0, rows1, acc_sh,
        ss0, ss1):
    c = lax.axis_index("c")
    s = lax.axis_index("s")
    w = c * _NSUB + s
    pltpu.sync_copy(z_hbm.at[pl.ds(s * _NPT, _NPT)],
                    acc_sh.at[pl.ds(s * _NPT, _NPT)])
    pltpu.sync_copy(dst_hbm.at[w], dstv)
    pltpu.sync_copy(ew_hbm.at[w], ewv)
    zrow = jnp.zeros((_H,), jnp.float32)
    bufs = (rows0, rows1)
    sems = (ss0, ss1)
    pend = [None, None]

    for rowsv in bufs:
      @pl.loop(0, _CB)
      def _(i):
        rowsv[i, :] = zrow

    plsc.subcore_barrier()
    iota = lax.iota(jnp.int32, 16)
    zer16 = jnp.zeros((16,), jnp.int32)

    for ch in range(_NCH):
      b = ch % 2
      rowsv = bufs[b]
      if pend[b] is not None:
        pend[b].wait()
      cful = jnp.full((16,), ch, jnp.int32)

      @pl.loop(0, _CB // 16)
      def _(g):
        ew16 = plsc.load_gather(ewv, [cful, iota + g * 16])
        plsc.store_scatter(rowsv, [iota + g * 16, zer16], ew16)

      pend[b] = pltpu.async_copy(rowsv, acc_sh.at[dstv.at[ch]],
                                 sems[b], add=True)

    for p in pend:
      p.wait()

    plsc.subcore_barrier()
    pltpu.sync_copy(acc_sh.at[pl.ds(s * _NPT, _NPT)],
                    out_hbm.at[c, pl.ds(s * _NPT, _NPT)])

  return k(dst3d, ew3d, zeros_nh)


def _sc_agg(src3d, dst3d, ew3d, table_nh, zeros_nh):
  @functools.partial(
      pl.kernel,
      out_type=jax.ShapeDtypeStruct((_NCORE, _NP, _H), jnp.float32),
      mesh=_mesh,
      compiler_params=_sc_cp,
      scratch_types=[
          pltpu.VMEM((_NCH, _CB), jnp.int32),
          pltpu.VMEM((_NCH, _CB), jnp.int32),
          pltpu.VMEM((_NCH, _CB), jnp.float32),
          pltpu.VMEM((_CB, _H), jnp.float32),
          pltpu.VMEM((_CB, _H), jnp.float32),
          pltpu.VMEM((_CB, _H), jnp.float32),
          pltpu.VMEM((_CB, _H), jnp.float32),
          pltpu.VMEM_SHARED((_NP, _H), jnp.float32),
          pltpu.SemaphoreType.DMA,
          pltpu.SemaphoreType.DMA,
          pltpu.SemaphoreType.DMA,
          pltpu.SemaphoreType.DMA,
          pltpu.SemaphoreType.DMA,
          pltpu.SemaphoreType.DMA,
          pltpu.SemaphoreType.DMA,
          pltpu.SemaphoreType.DMA,
      ],
  )
  def k(src_hbm, dst_hbm, ew_hbm, tab_hbm, z_hbm,
        out_hbm, srcv, dstv, ewv, rows0, rows1, rows2, rows3, acc_sh,
        gs0, gs1, gs2, gs3, ss0, ss1, ss2, ss3):
    c = lax.axis_index("c")
    s = lax.axis_index("s")
    w = c * _NSUB + s
    pltpu.sync_copy(z_hbm.at[pl.ds(s * _NPT, _NPT)],
                    acc_sh.at[pl.ds(s * _NPT, _NPT)])
    pltpu.sync_copy(src_hbm.at[w], srcv)
    pltpu.sync_copy(dst_hbm.at[w], dstv)
    pltpu.sync_copy(ew_hbm.at[w], ewv)
    plsc.subcore_barrier()

    bufs = (rows0, rows1, rows2, rows3)
    gsems = (gs0, gs1, gs2, gs3)
    ssems = (ss0, ss1, ss2, ss3)
    pg = [None] * 4
    ps = [None] * 4
    for g in range(3):
      pg[g] = pltpu.async_copy(tab_hbm.at[srcv.at[g]], bufs[g], gsems[g])
    for ch in range(_NCH):
      b = ch % 4
      rows = bufs[b]
      pg[b].wait()
      cful = jnp.full((16,), ch, jnp.int32)

      @pl.loop(0, _CB // 4)
      def _(q):
        i0 = q * 4
        for u in range(4):
          sp = plsc.load_gather(
              ewv, [cful, jnp.full((16,), i0 + u, jnp.int32)])
          rows[i0 + u, :] = rows[i0 + u, :] * sp

      ps[b] = pltpu.async_copy(rows, acc_sh.at[dstv.at[ch]],
                               ssems[b], add=True)
      g = ch + 3
      if g < _NCH:
        gb = g % 4
        if ps[gb] is not None:
          ps[gb].wait()
        pg[gb] = pltpu.async_copy(tab_hbm.at[srcv.at[g]], bufs[gb],
                                  gsems[gb])
    for ch in range(_NCH - 4, _NCH):
      ps[ch % 4].wait()

    plsc.subcore_barrier()
    pltpu.sync_copy(acc_sh.at[pl.ds(s * _NPT, _NPT)],
                    out_hbm.at[c, pl.ds(s * _NPT, _NPT)])

  return k(src3d, dst3d, ew3d, table_nh, zeros_nh)


_NPK = _NP // 8


def _tc_h1p(x4, W1p):
  def body(x_ref, w_ref, o_ref):
    acc = jnp.zeros((_NPK, 128), jnp.float32)
    for k in range(8):
      d = jnp.dot(x_ref[:, k, :], w_ref[...],
                  preferred_element_type=jnp.float32)
      acc = acc + (d if k == 0 else jnp.roll(d, 16 * k, axis=1))
    o_ref[...] = acc

  return pl.pallas_call(
      body,
      in_specs=[pl.BlockSpec((_NPK, 8, 128), lambda: (0, 0, 0)),
                pl.BlockSpec((128, 128), lambda: (0, 0))],
      out_specs=pl.BlockSpec((_NPK, 128), lambda: (0, 0)),
      out_shape=jax.ShapeDtypeStruct((_NPK, 128), jnp.float32))(x4, W1p)


def _tc_post_deg(partsp, h1p, S, B):
  def body(p_ref, h_ref, s_ref, b_ref, xs_ref, d_ref):
    deg8 = jnp.dot(p_ref[0] + p_ref[1], s_ref[...],
                   preferred_element_type=jnp.float32,
                   precision=lax.Precision.HIGHEST) + 1.0
    d = jnp.dot(lax.rsqrt(deg8), b_ref[...],
                preferred_element_type=jnp.float32,
                precision=lax.Precision.HIGHEST)
    d_ref[...] = d
    xs_ref[...] = h_ref[...] * d

  return pl.pallas_call(
      body,
      in_specs=[pl.BlockSpec((_NCORE, _NPK, 128), lambda: (0, 0, 0)),
                pl.BlockSpec((_NPK, 128), lambda: (0, 0)),
                pl.BlockSpec((128, 8), lambda: (0, 0)),
                pl.BlockSpec((8, 128), lambda: (0, 0))],
      out_specs=[pl.BlockSpec((_NPK, 128), lambda: (0, 0))] * 2,
      out_shape=[jax.ShapeDtypeStruct((_NPK, 128), jnp.float32)] * 2,
  )(partsp, h1p, S, B)


def _tc_mid(partsp, dinv, h1p, b1t):
  def body(p_ref, d_ref, h_ref, b_ref, rs_ref, r_ref):
    d = d_ref[...]
    a1 = d * (p_ref[0] + p_ref[1]) + d * d * h_ref[...] + b_ref[...]
    r = jnp.maximum(a1, 0.0)
    r_ref[...] = r
    rs_ref[...] = d * r

  return pl.pallas_call(
      body,
      in_specs=[pl.BlockSpec((_NCORE, _NPK, 128), lambda: (0, 0, 0)),
                pl.BlockSpec((_NPK, 128), lambda: (0, 0)),
                pl.BlockSpec((_NPK, 128), lambda: (0, 0)),
                pl.BlockSpec((1, 128), lambda: (0, 0))],
      out_specs=[pl.BlockSpec((_NPK, 128), lambda: (0, 0))] * 2,
      out_shape=[jax.ShapeDtypeStruct((_NPK, 128), jnp.float32)] * 2,
  )(partsp, dinv, h1p, b1t)


def _tc_final(partsp, dinv, rp, W2t, BT, P01, b2):
  def body(p_ref, d_ref, r_ref, w_ref, bt_ref, q_ref, b_ref, o_ref):
    d = d_ref[...]
    a2 = d * (p_ref[0] + p_ref[1]) + d * d * r_ref[...]
    bt = bt_ref[...]
    z0 = jnp.dot(a2 * w_ref[0], bt, preferred_element_type=jnp.float32,
                 precision=lax.Precision.HIGHEST) + b_ref[0, 0]
    z1 = jnp.dot(a2 * w_ref[1], bt, preferred_element_type=jnp.float32,
                 precision=lax.Precision.HIGHEST) + b_ref[0, 1]
    m = jnp.maximum(z0, z1)
    lse = m + jnp.log(jnp.exp(z0 - m) + jnp.exp(z1 - m))
    o_ref[...] = (jnp.dot(z0 - lse, q_ref[0],
                          preferred_element_type=jnp.float32,
                          precision=lax.Precision.HIGHEST)
                  + jnp.dot(z1 - lse, q_ref[1],
                            preferred_element_type=jnp.float32,
                            precision=lax.Precision.HIGHEST))

  return pl.pallas_call(
      body,
      in_specs=[pl.BlockSpec((_NCORE, _NPK, 128), lambda: (0, 0, 0)),
                pl.BlockSpec((_NPK, 128), lambda: (0, 0)),
                pl.BlockSpec((_NPK, 128), lambda: (0, 0)),
                pl.BlockSpec((2, 128), lambda: (0, 0)),
                pl.BlockSpec((128, 8), lambda: (0, 0)),
                pl.BlockSpec((2, 8, 16), lambda: (0, 0, 0)),
                pl.BlockSpec((1, 2), lambda: (0, 0))],
      out_specs=pl.BlockSpec((_NPK, 16), lambda: (0, 0)),
      out_shape=jax.ShapeDtypeStruct((_NPK, 16), jnp.float32),
  )(partsp, dinv, rp, W2t, BT, P01, b2)


def kernel(x, edge_index, edge_weight, W1, b1, W2, b2):
  src = edge_index[0].astype(jnp.int32)
  dst = edge_index[1].astype(jnp.int32)
  ew = edge_weight.astype(jnp.float32)
  pad = _EP - ew.shape[0]
  src3d = jnp.pad(src, (0, pad)).reshape(_NW, _NCH, _CB)
  dst3d = jnp.pad(dst, (0, pad)).reshape(_NW, _NCH, _CB)
  ew3d = jnp.pad(ew, (0, pad)).reshape(_NW, _NCH, _CB)
  zeros_nh = jnp.zeros((_NP, _H), jnp.float32)

  lanes = jnp.arange(128)
  S = (lanes[:, None] == jnp.arange(8)[None, :] * 16).astype(jnp.float32)
  B = (lanes[None, :] // 16 == jnp.arange(8)[:, None]).astype(jnp.float32)
  BT = B.T
  W1p = jnp.pad(W1, ((0, 0), (0, 128 - _H)))
  W2t = jnp.tile(W2.T, (1, 8))
  P01 = jnp.zeros((2, 8, 16), jnp.float32)
  for k in range(8):
    P01 = P01.at[0, k, 2 * k].set(1.0)
    P01 = P01.at[1, k, 2 * k + 1].set(1.0)
  b1t = jnp.tile(b1, 8).reshape(1, 128)
  b2t = b2.reshape(1, 2)

  xp = jnp.pad(x, ((0, _NP - x.shape[0]), (0, 0)))
  x4 = xp.reshape(_NPK, 8, 128)
  h1p = _tc_h1p(x4, W1p)
  degp = _sc_deg(dst3d, ew3d, zeros_nh)
  degpp = degp.reshape(_NCORE, _NPK, 128)
  xsp, dinv = _tc_post_deg(degpp, h1p, S, B)
  a1p = _sc_agg(src3d, dst3d, ew3d, xsp.reshape(_NP, _H), zeros_nh)
  rsp, rp = _tc_mid(a1p.reshape(_NCORE, _NPK, 128), dinv, h1p, b1t)
  a2p = _sc_agg(src3d, dst3d, ew3d, rsp.reshape(_NP, _H), zeros_nh)
  outp = _tc_final(a2p.reshape(_NCORE, _NPK, 128), dinv, rp, W2t, BT, P01, b2t)
  return outp.reshape(_NP, _C)[:_N]
